# baseline (speedup 1.0000x reference)
import functools

import jax
import jax.numpy as jnp
from jax import lax
from jax.experimental import pallas as pl
from jax.experimental.pallas import tpu as pltpu
from jax.experimental.pallas import tpu_sc as plsc

_B, _S, _D = 4, 8192, 1024
_NC, _NS = 2, 16
_NW = _NC * _NS
_SPW = _S // _NW
_CH = 16
_CHW = _CH * _D
_UNROLL = 8

_mesh = plsc.VectorSubcoreMesh(core_axis_name="c", subcore_axis_name="s")


@functools.partial(
    pl.kernel,
    mesh=_mesh,
    out_type=jax.ShapeDtypeStruct((_B * _S * _D,), jnp.float32),
    scratch_types=[
        pltpu.VMEM((_CHW,), jnp.float32),
        pltpu.VMEM((_CHW,), jnp.float32),
    ],
)
def _sc_add(x_hbm, pe_hbm, out_hbm, pe_v, x_v):
    wid = lax.axis_index("s") * _NC + lax.axis_index("c")
    s_base = wid * _SPW

    def chunk_body(ci, carry):
        row0 = s_base + ci * _CH
        pltpu.sync_copy(pe_hbm.at[pl.ds(row0 * _D, _CHW)], pe_v)

        def b_body(b, carry):
            off = (b * _S + row0) * _D
            pltpu.sync_copy(x_hbm.at[pl.ds(off, _CHW)], x_v)

            def add_body(i, carry):
                base = i * 16 * _UNROLL
                for u in range(_UNROLL):
                    sl = pl.ds(base + u * 16, 16)
                    x_v[sl] = x_v[sl] + pe_v[sl]
                return carry

            lax.fori_loop(0, _CHW // (16 * _UNROLL), add_body, 0)
            pltpu.sync_copy(x_v, out_hbm.at[pl.ds(off, _CHW)])
            return carry

        lax.fori_loop(0, _B, b_body, 0)
        return carry

    lax.fori_loop(0, _SPW // _CH, chunk_body, 0)


def kernel(x, pe_table):
    B, S, D = x.shape
    out = _sc_add(x.reshape(-1), pe_table[:S].reshape(-1))
    return out.reshape(B, S, D)


# speedup vs baseline: 1.3071x; 1.3071x over previous
import functools

import jax
import jax.numpy as jnp
from jax import lax
from jax.experimental import pallas as pl
from jax.experimental.pallas import tpu as pltpu
from jax.experimental.pallas import tpu_sc as plsc

_B, _S, _D = 4, 8192, 1024
_NC, _NS = 2, 16
_NW = _NC * _NS
_SPW = _S // _NW
_CH = 16
_CHW = _CH * _D
_NCHUNK = _SPW // _CH
_UNROLL = 8

_mesh = plsc.VectorSubcoreMesh(core_axis_name="c", subcore_axis_name="s")


@functools.partial(
    pl.kernel,
    mesh=_mesh,
    out_type=jax.ShapeDtypeStruct((_B * _S * _D,), jnp.float32),
    scratch_types=(
        [pltpu.VMEM((_CHW,), jnp.float32) for _ in range(4)]
        + [pltpu.VMEM((_CHW,), jnp.float32) for _ in range(2)]
        + [pltpu.SemaphoreType.DMA for _ in range(10)]
    ),
)
def _sc_add(x_hbm, pe_hbm, out_hbm,
            xb0, xb1, xb2, xb3, peb0, peb1,
            si0, si1, si2, si3, so0, so1, so2, so3, sp0, sp1):
    x_bufs = [xb0, xb1, xb2, xb3]
    pe_bufs = [peb0, peb1]
    in_sems = [si0, si1, si2, si3]
    out_sems = [so0, so1, so2, so3]
    pe_sems = [sp0, sp1]

    wid = lax.axis_index("s") * _NC + lax.axis_index("c")
    s_base = wid * _SPW

    steps = [(c, b) for c in range(_NCHUNK) for b in range(_B)]

    def x_off(c, b):
        return (b * _S + s_base + c * _CH) * _D

    def pe_off(c):
        return (s_base + c * _CH) * _D

    def load_x(c, b):
        return pltpu.async_copy(
            x_hbm.at[pl.ds(x_off(c, b), _CHW)], x_bufs[b], in_sems[b])

    def load_pe(c):
        return pltpu.async_copy(
            pe_hbm.at[pl.ds(pe_off(c), _CHW)], pe_bufs[c % 2], pe_sems[c % 2])

    def store_x(c, b):
        return pltpu.async_copy(
            x_bufs[b], out_hbm.at[pl.ds(x_off(c, b), _CHW)], out_sems[b])

    h_pe = [load_pe(0), None]
    h_in = [load_x(0, 0), load_x(0, 1), load_x(0, 2), None]
    h_out = [None, None, None, None]

    for g, (c, b) in enumerate(steps):
        h_in[b].wait()
        if b == 0:
            h_pe[c % 2].wait()
        if b == 1 and c + 1 < _NCHUNK:
            h_pe[(c + 1) % 2] = load_pe(c + 1)
        pe_v = pe_bufs[c % 2]
        x_v = x_bufs[b]

        def add_body(i, carry, x_v=x_v, pe_v=pe_v):
            base = i * 16 * _UNROLL
            for u in range(_UNROLL):
                sl = pl.ds(base + u * 16, 16)
                x_v[sl] = x_v[sl] + pe_v[sl]
            return carry

        lax.fori_loop(0, _CHW // (16 * _UNROLL), add_body, 0)

        h_out[b] = store_x(c, b)
        if g + 3 < len(steps):
            nc, nb = steps[g + 3]
            if h_out[nb] is not None:
                h_out[nb].wait()
            h_in[nb] = load_x(nc, nb)

    for b in range(4):
        h_out[b].wait()


def kernel(x, pe_table):
    B, S, D = x.shape
    out = _sc_add(x.reshape(-1), pe_table[:S].reshape(-1))
    return out.reshape(B, S, D)
